# baseline (speedup 1.0000x reference)
import functools

import jax
import jax.numpy as jnp
from jax import lax
from jax.experimental import pallas as pl
from jax.experimental.pallas import tpu as pltpu
from jax.experimental.pallas import tpu_sc as plsc

NN = 10000
EE = 320000
NPAD = 10112
ROWS_PER_TILE = 632
NWORK = 32
EPW = 10240
EPAD = NWORK * EPW
W1ROW = 144
W2ROW = 64
BL = 4


def _tc_layer1(xp, W1, AS, AD):
    def body(x_ref, w_ref, as_ref, ad_ref, xs_ref, d_ref):
        xw = jnp.dot(x_ref[...], w_ref[...], preferred_element_type=jnp.float32)
        s = jnp.dot(xw, as_ref[...], preferred_element_type=jnp.float32)
        d = jnp.dot(xw, ad_ref[...], preferred_element_type=jnp.float32)
        xs_ref[...] = jnp.concatenate([xw, s], axis=1)
        d_ref[...] = d

    return pl.pallas_call(
        body,
        grid=(8,),
        in_specs=[
            pl.BlockSpec((NPAD // 8, 128), lambda i: (i, 0)),
            pl.BlockSpec((128, 128), lambda i: (0, 0)),
            pl.BlockSpec((128, 16), lambda i: (0, 0)),
            pl.BlockSpec((128, 16), lambda i: (0, 0)),
        ],
        out_specs=[
            pl.BlockSpec((NPAD // 8, W1ROW), lambda i: (i, 0)),
            pl.BlockSpec((NPAD // 8, 16), lambda i: (i, 0)),
        ],
        out_shape=[
            jax.ShapeDtypeStruct((NPAD, W1ROW), jnp.float32),
            jax.ShapeDtypeStruct((NPAD, 16), jnp.float32),
        ],
    )(xp, W1, AS, AD)


def _pack_idx(srcp, dstp, kch):
    s = srcp.reshape(-1, 1, kch)
    d = dstp.reshape(-1, 1, kch)
    return jnp.concatenate([s, d], axis=1)


def _sc_edge_phase(xs, d1, eidx_arr, row_w, msg_groups, den_col,
                   per_group_w, kch, zrows, nc0, nc1):
    mesh = plsc.VectorSubcoreMesh(core_axis_name="c", subcore_axis_name="s")
    assert 16 * (nc0 + nc1) * kch == EPAD and nc0 % 8 == 0 and nc1 % 8 == 0 and nc1 >= 0

    vm = pltpu.VMEM
    @functools.partial(
        pl.kernel,
        out_type=[jax.ShapeDtypeStruct((NPAD, row_w), jnp.float32),
                  jax.ShapeDtypeStruct((NPAD, row_w), jnp.float32)],
        mesh=mesh,
        compiler_params=pltpu.CompilerParams(use_tc_tiling_on_sc=False),
        scratch_types=[
            vm((BL, 2, kch), jnp.int32), vm((BL, 2, kch), jnp.int32),
            vm((kch, row_w), jnp.float32), vm((kch, row_w), jnp.float32),
            vm((kch, 16), jnp.float32), vm((kch, 16), jnp.float32),
            vm((kch, row_w), jnp.float32), vm((kch, row_w), jnp.float32),
            pltpu.VMEM_SHARED((NPAD, row_w), jnp.float32),
            pltpu.SemaphoreType.DMA, pltpu.SemaphoreType.DMA,
            pltpu.SemaphoreType.DMA, pltpu.SemaphoreType.DMA,
            pltpu.SemaphoreType.DMA, pltpu.SemaphoreType.DMA,
        ],
    )
    def k(xs_hbm, d_hbm, eidx_hbm, z_hbm, out0_hbm, out1_hbm,
          ib0, ib1, xsb0, xsb1, db0, db1, cb0, cb1, accum,
          sg0, sg1, ss0, ss1, si0, si1):
        cid = lax.axis_index("c")
        sid = lax.axis_index("s")
        cbase = jnp.where(cid == 0, sid * nc0, 16 * nc0 + sid * nc1)
        nblk = jnp.where(cid == 0, nc0 // BL, nc1 // BL)

        ibs = (ib0, ib1)
        xsb = (xsb0, xsb1)
        db = (db0, db1)
        comb = (cb0, cb1)
        sg = (sg0, sg1)
        ss = (ss0, ss1)
        si = (si0, si1)

        pltpu.sync_copy(z_hbm, accum.at[pl.ds(sid * ROWS_PER_TILE, ROWS_PER_TILE)])
        plsc.subcore_barrier()

        def issue_gathers(idxref, ii, b):
            pltpu.async_copy(xs_hbm.at[idxref.at[ii, 0]], xsb[b], sg[b])
            pltpu.async_copy(d_hbm.at[idxref.at[ii, 1]], db[b], sg[b])

        def wait_gathers(b):
            pltpu.make_async_copy(xs_hbm.at[pl.ds(0, kch)], xsb[b], sg[b]).wait()
            pltpu.make_async_copy(d_hbm.at[pl.ds(0, kch)], db[b], sg[b]).wait()

        def wait_scatter(b):
            pltpu.make_async_copy(xs_hbm.at[pl.ds(0, kch)], comb[b], ss[b]).wait()

        def wait_iblk(b2):
            pltpu.make_async_copy(eidx_hbm.at[pl.ds(0, BL)], ibs[b2], si[b2]).wait()

        @pl.when(nblk > 0)
        def _():
            pltpu.sync_copy(eidx_hbm.at[pl.ds(cbase, BL)], ib0)
            issue_gathers(ib0, 0, 0)

        def blockpair(t2, carry):
            for b2 in range(2):
                t = 2 * t2 + b2
                ib, ibn = ibs[b2], ibs[1 - b2]
                for i in range(BL):
                    p = i % 2
                    jg = t * BL + i
                    @pl.when(jg >= 1)
                    def _():
                        wait_scatter(1 - p)

                    if i == 0:
                        @pl.when(t + 1 < nblk)
                        def _():
                            pltpu.async_copy(
                                eidx_hbm.at[pl.ds(cbase + (t + 1) * BL, BL)],
                                ibn, si[1 - b2])
                    if i < BL - 1:
                        issue_gathers(ib, i + 1, 1 - p)
                    else:
                        @pl.when(t + 1 < nblk)
                        def _():
                            wait_iblk(1 - b2)
                            issue_gathers(ibn, 0, 1 - p)

                    wait_gathers(p)
                    xsbb, dbb, combb = xsb[p], db[p], comb[p]

                    @plsc.parallel_loop(0, kch, step=1, unroll=4)
                    def _edges(e):
                        sv = xsbb[e, pl.ds(den_col, 16)] + dbb[e, :]
                        w = jnp.exp(jnp.maximum(sv, 0.2 * sv))
                        combb[e, pl.ds(den_col, 16)] = w
                        for g in range(msg_groups):
                            wsc = w[g] if per_group_w else w[0]
                            combb[e, pl.ds(16 * g, 16)] = xsbb[e, pl.ds(16 * g, 16)] * wsc
                    pltpu.async_copy(comb[p], accum.at[ib.at[i, 1]], ss[p], add=True)
            return carry

        lax.fori_loop(0, nblk // 2, blockpair, 0, unroll=False)

        @pl.when(nblk > 0)
        def _():
            wait_scatter(1)
        plsc.subcore_barrier()
        @pl.when(cid == 0)
        def _():
            pltpu.sync_copy(accum.at[pl.ds(sid * ROWS_PER_TILE, ROWS_PER_TILE)],
                            out0_hbm.at[pl.ds(sid * ROWS_PER_TILE, ROWS_PER_TILE)])

        @pl.when(cid == 1)
        def _():
            pltpu.sync_copy(accum.at[pl.ds(sid * ROWS_PER_TILE, ROWS_PER_TILE)],
                            out1_hbm.at[pl.ds(sid * ROWS_PER_TILE, ROWS_PER_TILE)])

    return k(xs, d1, eidx_arr, zrows)


def _tc_combine1(p0, p1, b1, W2p, AS2, AD2):
    def body(p0_ref, p1_ref, b_ref, w_ref, as_ref, ad_ref, xs_ref, d_ref):
        num = p0_ref[...] + p1_ref[...]
        msg = num[:, 0:128]
        recip = 1.0 / (num[:, 128:136] + 1e-16)
        parts = [msg[:, 16 * h:16 * h + 16] * recip[:, h:h + 1] for h in range(8)]
        h1 = jnp.concatenate(parts, axis=1) + b_ref[...]
        h1 = jnp.where(h1 > 0, h1, jnp.exp(jnp.minimum(h1, 0.0)) - 1.0)
        xw2 = jnp.dot(h1, w_ref[...], preferred_element_type=jnp.float32)
        s2 = jnp.dot(xw2, as_ref[...], preferred_element_type=jnp.float32)
        d2 = jnp.dot(xw2, ad_ref[...], preferred_element_type=jnp.float32)
        xs_ref[...] = jnp.concatenate([xw2, s2], axis=1)
        d_ref[...] = d2

    return pl.pallas_call(
        body,
        grid=(8,),
        in_specs=[
            pl.BlockSpec((NPAD // 8, W1ROW), lambda i: (i, 0)),
            pl.BlockSpec((NPAD // 8, W1ROW), lambda i: (i, 0)),
            pl.BlockSpec((1, 128), lambda i: (0, 0)),
            pl.BlockSpec((128, 48), lambda i: (0, 0)),
            pl.BlockSpec((48, 16), lambda i: (0, 0)),
            pl.BlockSpec((48, 16), lambda i: (0, 0)),
        ],
        out_specs=[
            pl.BlockSpec((NPAD // 8, W2ROW), lambda i: (i, 0)),
            pl.BlockSpec((NPAD // 8, 16), lambda i: (i, 0)),
        ],
        out_shape=[
            jax.ShapeDtypeStruct((NPAD, W2ROW), jnp.float32),
            jax.ShapeDtypeStruct((NPAD, 16), jnp.float32),
        ],
    )(p0, p1, b1, W2p, AS2, AD2)


def _tc_final(q0, q1, b2):
    def body(q0_ref, q1_ref, b_ref, o_ref):
        num = q0_ref[...] + q1_ref[...]
        z = num[:, 0:40] / (num[:, 48:49] + 1e-16) + b_ref[...]
        z = jnp.where(z > 0, z, jnp.exp(jnp.minimum(z, 0.0)) - 1.0)
        m = jnp.max(z, axis=1, keepdims=True)
        ez = jnp.exp(z - m)
        ssum = jnp.sum(ez, axis=1, keepdims=True)
        o_ref[...] = z - m - jnp.log(ssum)

    return pl.pallas_call(
        body,
        grid=(8,),
        in_specs=[
            pl.BlockSpec((NPAD // 8, W2ROW), lambda i: (i, 0)),
            pl.BlockSpec((NPAD // 8, W2ROW), lambda i: (i, 0)),
            pl.BlockSpec((1, 40), lambda i: (0, 0)),
        ],
        out_specs=pl.BlockSpec((NPAD // 8, 40), lambda i: (i, 0)),
        out_shape=jax.ShapeDtypeStruct((NPAD, 40), jnp.float32),
    )(q0, q1, b2)


@jax.jit
def kernel(x, edge_index, W1, att_src1, att_dst1, bias1, W2, att_src2, att_dst2, bias2):
    f32 = jnp.float32
    xp = jnp.zeros((NPAD, 128), f32).at[:NN].set(x)
    src = edge_index[0].astype(jnp.int32)
    dst = edge_index[1].astype(jnp.int32)
    padn = jnp.full((EPAD - EE,), NN, jnp.int32)
    srcp = jnp.concatenate([src, padn])
    dstp = jnp.concatenate([dst, padn])

    blkmask = (jnp.arange(128)[:, None] // 16 == jnp.arange(16)[None, :])
    AS = jnp.where(blkmask, att_src1.reshape(128)[:, None], 0.0)
    AD = jnp.where(blkmask, att_dst1.reshape(128)[:, None], 0.0)
    W2p = jnp.pad(W2, ((0, 0), (0, 8)))
    col0 = (jnp.arange(16)[None, :] == 0) & (jnp.arange(48)[:, None] < 40)
    a2pad = jnp.pad(att_src2.reshape(40), (0, 8))
    d2pad = jnp.pad(att_dst2.reshape(40), (0, 8))
    AS2 = jnp.where(col0, a2pad[:, None], 0.0)
    AD2 = jnp.where(col0, d2pad[:, None], 0.0)
    b1 = bias1.reshape(1, 128)
    b2 = bias2.reshape(1, 40)

    z1 = jnp.zeros((ROWS_PER_TILE, W1ROW), f32)
    z2 = jnp.zeros((ROWS_PER_TILE, W2ROW), f32)

    xs1, d1 = _tc_layer1(xp, W1, AS, AD)
    p0, p1 = _sc_edge_phase(xs1, d1, _pack_idx(srcp, dstp, 64), W1ROW, 8, 128,
                            True, 64, z1, 320, 0)
    xs2, d2 = _tc_combine1(p0, p1, b1, W2p, AS2, AD2)
    q0, q1 = _sc_edge_phase(xs2, d2, _pack_idx(srcp, dstp, 128), W2ROW, 3, 48,
                            False, 128, z2, 160, 0)
    out = _tc_final(q0, q1, b2)
    return out[:NN]


# speedup vs baseline: 1.2412x; 1.2412x over previous
import functools

import jax
import jax.numpy as jnp
from jax import lax
from jax.experimental import pallas as pl
from jax.experimental.pallas import tpu as pltpu
from jax.experimental.pallas import tpu_sc as plsc

NN = 10000
EE = 320000
NPAD = 10112
ROWS_PER_TILE = 632
NWORK = 32
EPW = 10240
EPAD = NWORK * EPW
W1ROW = 144
W2ROW = 64
BL = 4


def _tc_layer1(xp, W1, AS, AD):
    def body(x_ref, w_ref, as_ref, ad_ref, xs_ref, d_ref):
        xw = jnp.dot(x_ref[...], w_ref[...], preferred_element_type=jnp.float32)
        s = jnp.dot(xw, as_ref[...], preferred_element_type=jnp.float32)
        d = jnp.dot(xw, ad_ref[...], preferred_element_type=jnp.float32)
        xs_ref[...] = jnp.concatenate([xw, s], axis=1)
        d_ref[...] = d

    return pl.pallas_call(
        body,
        grid=(8,),
        in_specs=[
            pl.BlockSpec((NPAD // 8, 128), lambda i: (i, 0)),
            pl.BlockSpec((128, 128), lambda i: (0, 0)),
            pl.BlockSpec((128, 16), lambda i: (0, 0)),
            pl.BlockSpec((128, 16), lambda i: (0, 0)),
        ],
        out_specs=[
            pl.BlockSpec((NPAD // 8, W1ROW), lambda i: (i, 0)),
            pl.BlockSpec((NPAD // 8, 16), lambda i: (i, 0)),
        ],
        out_shape=[
            jax.ShapeDtypeStruct((NPAD, W1ROW), jnp.float32),
            jax.ShapeDtypeStruct((NPAD, 16), jnp.float32),
        ],
    )(xp, W1, AS, AD)


def _pack_idx(srcp, dstp, kch):
    s = srcp.reshape(-1, 1, kch)
    d = dstp.reshape(-1, 1, kch)
    return jnp.concatenate([s, d], axis=1)


def _sc_edge_phase(xs, d1, eidx_arr, row_w, msg_groups, den_col,
                   per_group_w, kch, zrows, nc0, nc1):
    mesh = plsc.VectorSubcoreMesh(core_axis_name="c", subcore_axis_name="s")
    assert 16 * (nc0 + nc1) * kch == EPAD and nc0 % 8 == 0 and nc1 % 8 == 0 and nc1 >= 0

    vm = pltpu.VMEM
    @functools.partial(
        pl.kernel,
        out_type=[jax.ShapeDtypeStruct((NPAD, row_w), jnp.float32),
                  jax.ShapeDtypeStruct((NPAD, row_w), jnp.float32)],
        mesh=mesh,
        compiler_params=pltpu.CompilerParams(use_tc_tiling_on_sc=False),
        scratch_types=[
            vm((BL, 2, kch), jnp.int32), vm((BL, 2, kch), jnp.int32),
            vm((kch, row_w), jnp.float32), vm((kch, row_w), jnp.float32),
            vm((kch, 16), jnp.float32), vm((kch, 16), jnp.float32),
            vm((kch, row_w), jnp.float32), vm((kch, row_w), jnp.float32),
            pltpu.VMEM_SHARED((NPAD, row_w), jnp.float32),
            pltpu.SemaphoreType.DMA, pltpu.SemaphoreType.DMA,
            pltpu.SemaphoreType.DMA, pltpu.SemaphoreType.DMA,
            pltpu.SemaphoreType.DMA, pltpu.SemaphoreType.DMA,
        ],
    )
    def k(xs_hbm, d_hbm, eidx_hbm, z_hbm, out0_hbm, out1_hbm,
          ib0, ib1, xsb0, xsb1, db0, db1, cb0, cb1, accum,
          sg0, sg1, ss0, ss1, si0, si1):
        cid = lax.axis_index("c")
        sid = lax.axis_index("s")
        cbase = jnp.where(cid == 0, sid * nc0, 16 * nc0 + sid * nc1)
        nblk = jnp.where(cid == 0, nc0 // BL, nc1 // BL)

        ibs = (ib0, ib1)
        xsb = (xsb0, xsb1)
        db = (db0, db1)
        comb = (cb0, cb1)
        sg = (sg0, sg1)
        ss = (ss0, ss1)
        si = (si0, si1)

        pltpu.sync_copy(z_hbm, accum.at[pl.ds(sid * ROWS_PER_TILE, ROWS_PER_TILE)])
        plsc.subcore_barrier()

        def issue_gathers(idxref, ii, b):
            pltpu.async_copy(xs_hbm.at[idxref.at[ii, 0]], xsb[b], sg[b])
            pltpu.async_copy(d_hbm.at[idxref.at[ii, 1]], db[b], sg[b])

        def wait_gathers(b):
            pltpu.make_async_copy(xs_hbm.at[pl.ds(0, kch)], xsb[b], sg[b]).wait()
            pltpu.make_async_copy(d_hbm.at[pl.ds(0, kch)], db[b], sg[b]).wait()

        def wait_scatter(b):
            pltpu.make_async_copy(xs_hbm.at[pl.ds(0, kch)], comb[b], ss[b]).wait()

        def wait_iblk(b2):
            pltpu.make_async_copy(eidx_hbm.at[pl.ds(0, BL)], ibs[b2], si[b2]).wait()

        @pl.when(nblk > 0)
        def _():
            pltpu.sync_copy(eidx_hbm.at[pl.ds(cbase, BL)], ib0)
            issue_gathers(ib0, 0, 0)

        def blockpair(t2, carry):
            for b2 in range(2):
                t = 2 * t2 + b2
                ib, ibn = ibs[b2], ibs[1 - b2]
                for i in range(BL):
                    p = i % 2
                    jg = t * BL + i
                    @pl.when(jg >= 1)
                    def _():
                        wait_scatter(1 - p)

                    if i == 0:
                        @pl.when(t + 1 < nblk)
                        def _():
                            pltpu.async_copy(
                                eidx_hbm.at[pl.ds(cbase + (t + 1) * BL, BL)],
                                ibn, si[1 - b2])
                    if i < BL - 1:
                        issue_gathers(ib, i + 1, 1 - p)
                    else:
                        @pl.when(t + 1 < nblk)
                        def _():
                            wait_iblk(1 - b2)
                            issue_gathers(ibn, 0, 1 - p)

                    wait_gathers(p)
                    xsbb, dbb, combb = xsb[p], db[p], comb[p]

                    @plsc.parallel_loop(0, kch, step=1, unroll=4)
                    def _edges(e):
                        sv = xsbb[e, pl.ds(den_col, 16)] + dbb[e, :]
                        w = jnp.exp(jnp.maximum(sv, 0.2 * sv))
                        combb[e, pl.ds(den_col, 16)] = w
                        for g in range(msg_groups):
                            wsc = w[g] if per_group_w else w[0]
                            combb[e, pl.ds(16 * g, 16)] = xsbb[e, pl.ds(16 * g, 16)] * wsc
                    pltpu.async_copy(comb[p], accum.at[ib.at[i, 1]], ss[p], add=True)
            return carry

        lax.fori_loop(0, nblk // 2, blockpair, 0, unroll=False)

        @pl.when(nblk > 0)
        def _():
            wait_scatter(1)
        plsc.subcore_barrier()
        @pl.when(cid == 0)
        def _():
            pltpu.sync_copy(accum.at[pl.ds(sid * ROWS_PER_TILE, ROWS_PER_TILE)],
                            out0_hbm.at[pl.ds(sid * ROWS_PER_TILE, ROWS_PER_TILE)])

        @pl.when(cid == 1)
        def _():
            pltpu.sync_copy(accum.at[pl.ds(sid * ROWS_PER_TILE, ROWS_PER_TILE)],
                            out1_hbm.at[pl.ds(sid * ROWS_PER_TILE, ROWS_PER_TILE)])

    return k(xs, d1, eidx_arr, zrows)


def _tc_combine1(p0, p1, b1, W2p, AS2, AD2):
    def body(p0_ref, p1_ref, b_ref, w_ref, as_ref, ad_ref, xs_ref, d_ref):
        num = p0_ref[...] + p1_ref[...]
        msg = num[:, 0:128]
        recip = 1.0 / (num[:, 128:136] + 1e-16)
        parts = [msg[:, 16 * h:16 * h + 16] * recip[:, h:h + 1] for h in range(8)]
        h1 = jnp.concatenate(parts, axis=1) + b_ref[...]
        h1 = jnp.where(h1 > 0, h1, jnp.exp(jnp.minimum(h1, 0.0)) - 1.0)
        xw2 = jnp.dot(h1, w_ref[...], preferred_element_type=jnp.float32)
        s2 = jnp.dot(xw2, as_ref[...], preferred_element_type=jnp.float32)
        d2 = jnp.dot(xw2, ad_ref[...], preferred_element_type=jnp.float32)
        xs_ref[...] = jnp.concatenate([xw2, s2], axis=1)
        d_ref[...] = d2

    return pl.pallas_call(
        body,
        grid=(8,),
        in_specs=[
            pl.BlockSpec((NPAD // 8, W1ROW), lambda i: (i, 0)),
            pl.BlockSpec((NPAD // 8, W1ROW), lambda i: (i, 0)),
            pl.BlockSpec((1, 128), lambda i: (0, 0)),
            pl.BlockSpec((128, 48), lambda i: (0, 0)),
            pl.BlockSpec((48, 16), lambda i: (0, 0)),
            pl.BlockSpec((48, 16), lambda i: (0, 0)),
        ],
        out_specs=[
            pl.BlockSpec((NPAD // 8, W2ROW), lambda i: (i, 0)),
            pl.BlockSpec((NPAD // 8, 16), lambda i: (i, 0)),
        ],
        out_shape=[
            jax.ShapeDtypeStruct((NPAD, W2ROW), jnp.float32),
            jax.ShapeDtypeStruct((NPAD, 16), jnp.float32),
        ],
    )(p0, p1, b1, W2p, AS2, AD2)


def _tc_final(q0, q1, b2):
    def body(q0_ref, q1_ref, b_ref, o_ref):
        num = q0_ref[...] + q1_ref[...]
        z = num[:, 0:40] / (num[:, 48:49] + 1e-16) + b_ref[...]
        z = jnp.where(z > 0, z, jnp.exp(jnp.minimum(z, 0.0)) - 1.0)
        m = jnp.max(z, axis=1, keepdims=True)
        ez = jnp.exp(z - m)
        ssum = jnp.sum(ez, axis=1, keepdims=True)
        o_ref[...] = z - m - jnp.log(ssum)

    return pl.pallas_call(
        body,
        grid=(8,),
        in_specs=[
            pl.BlockSpec((NPAD // 8, W2ROW), lambda i: (i, 0)),
            pl.BlockSpec((NPAD // 8, W2ROW), lambda i: (i, 0)),
            pl.BlockSpec((1, 40), lambda i: (0, 0)),
        ],
        out_specs=pl.BlockSpec((NPAD // 8, 40), lambda i: (i, 0)),
        out_shape=jax.ShapeDtypeStruct((NPAD, 40), jnp.float32),
    )(q0, q1, b2)


@jax.jit
def kernel(x, edge_index, W1, att_src1, att_dst1, bias1, W2, att_src2, att_dst2, bias2):
    f32 = jnp.float32
    xp = jnp.zeros((NPAD, 128), f32).at[:NN].set(x)
    src = edge_index[0].astype(jnp.int32)
    dst = edge_index[1].astype(jnp.int32)
    padn = jnp.full((EPAD - EE,), NN, jnp.int32)
    srcp = jnp.concatenate([src, padn])
    dstp = jnp.concatenate([dst, padn])

    blkmask = (jnp.arange(128)[:, None] // 16 == jnp.arange(16)[None, :])
    AS = jnp.where(blkmask, att_src1.reshape(128)[:, None], 0.0)
    AD = jnp.where(blkmask, att_dst1.reshape(128)[:, None], 0.0)
    W2p = jnp.pad(W2, ((0, 0), (0, 8)))
    col0 = (jnp.arange(16)[None, :] == 0) & (jnp.arange(48)[:, None] < 40)
    a2pad = jnp.pad(att_src2.reshape(40), (0, 8))
    d2pad = jnp.pad(att_dst2.reshape(40), (0, 8))
    AS2 = jnp.where(col0, a2pad[:, None], 0.0)
    AD2 = jnp.where(col0, d2pad[:, None], 0.0)
    b1 = bias1.reshape(1, 128)
    b2 = bias2.reshape(1, 40)

    z1 = jnp.zeros((ROWS_PER_TILE, W1ROW), f32)
    z2 = jnp.zeros((ROWS_PER_TILE, W2ROW), f32)

    xs1, d1 = _tc_layer1(xp, W1, AS, AD)
    p0, p1 = _sc_edge_phase(xs1, d1, _pack_idx(srcp, dstp, 64), W1ROW, 8, 128,
                            True, 64, z1, 224, 96)
    xs2, d2 = _tc_combine1(p0, p1, b1, W2p, AS2, AD2)
    q0, q1 = _sc_edge_phase(xs2, d2, _pack_idx(srcp, dstp, 128), W2ROW, 3, 48,
                            False, 128, z2, 112, 48)
    out = _tc_final(q0, q1, b2)
    return out[:NN]


# speedup vs baseline: 1.4017x; 1.1293x over previous
import functools

import jax
import jax.numpy as jnp
from jax import lax
from jax.experimental import pallas as pl
from jax.experimental.pallas import tpu as pltpu
from jax.experimental.pallas import tpu_sc as plsc

NN = 10000
EE = 320000
NPAD = 10112
ROWS_PER_TILE = 632
NWORK = 32
EPW = 10240
EPAD = NWORK * EPW
W1ROW = 144
W2ROW = 64
BL = 4


def _tc_layer1(xp, W1, AS, AD):
    def body(x_ref, w_ref, as_ref, ad_ref, xs_ref, d_ref):
        xw = jnp.dot(x_ref[...], w_ref[...], preferred_element_type=jnp.float32)
        s = jnp.dot(xw, as_ref[...], preferred_element_type=jnp.float32)
        d = jnp.dot(xw, ad_ref[...], preferred_element_type=jnp.float32)
        xs_ref[...] = jnp.concatenate([xw, s], axis=1)
        d_ref[...] = d

    return pl.pallas_call(
        body,
        grid=(8,),
        in_specs=[
            pl.BlockSpec((NPAD // 8, 128), lambda i: (i, 0)),
            pl.BlockSpec((128, 128), lambda i: (0, 0)),
            pl.BlockSpec((128, 16), lambda i: (0, 0)),
            pl.BlockSpec((128, 16), lambda i: (0, 0)),
        ],
        out_specs=[
            pl.BlockSpec((NPAD // 8, W1ROW), lambda i: (i, 0)),
            pl.BlockSpec((NPAD // 8, 16), lambda i: (i, 0)),
        ],
        out_shape=[
            jax.ShapeDtypeStruct((NPAD, W1ROW), jnp.float32),
            jax.ShapeDtypeStruct((NPAD, 16), jnp.float32),
        ],
    )(xp, W1, AS, AD)


def _pack_idx(srcp, dstp, kch):
    s = srcp.reshape(-1, 1, kch)
    d = dstp.reshape(-1, 1, kch)
    return jnp.concatenate([s, d], axis=1)


def _sc_edge_phase(xs, d1, eidx_arr, row_w, msg_groups, den_col,
                   per_group_w, kch, zrows, nc0, nc1):
    mesh = plsc.VectorSubcoreMesh(core_axis_name="c", subcore_axis_name="s")
    assert 16 * (nc0 + nc1) * kch == EPAD and nc0 % 8 == 0 and nc1 % 8 == 0 and nc1 >= 0

    vm = pltpu.VMEM
    @functools.partial(
        pl.kernel,
        out_type=[jax.ShapeDtypeStruct((NPAD, row_w), jnp.float32),
                  jax.ShapeDtypeStruct((NPAD, row_w), jnp.float32)],
        mesh=mesh,
        compiler_params=pltpu.CompilerParams(use_tc_tiling_on_sc=False),
        scratch_types=[
            vm((BL, 2, kch), jnp.int32), vm((BL, 2, kch), jnp.int32),
            vm((kch, row_w), jnp.float32), vm((kch, row_w), jnp.float32),
            vm((kch, 16), jnp.float32), vm((kch, 16), jnp.float32),
            vm((kch, row_w), jnp.float32), vm((kch, row_w), jnp.float32),
            pltpu.VMEM_SHARED((NPAD, row_w), jnp.float32),
            pltpu.SemaphoreType.DMA, pltpu.SemaphoreType.DMA,
            pltpu.SemaphoreType.DMA, pltpu.SemaphoreType.DMA,
            pltpu.SemaphoreType.DMA, pltpu.SemaphoreType.DMA,
        ],
    )
    def k(xs_hbm, d_hbm, eidx_hbm, z_hbm, out0_hbm, out1_hbm,
          ib0, ib1, xsb0, xsb1, db0, db1, cb0, cb1, accum,
          sg0, sg1, ss0, ss1, si0, si1):
        cid = lax.axis_index("c")
        sid = lax.axis_index("s")
        cbase = jnp.where(cid == 0, sid * nc0, 16 * nc0 + sid * nc1)
        nblk = jnp.where(cid == 0, nc0 // BL, nc1 // BL)

        ibs = (ib0, ib1)
        xsb = (xsb0, xsb1)
        db = (db0, db1)
        comb = (cb0, cb1)
        sg = (sg0, sg1)
        ss = (ss0, ss1)
        si = (si0, si1)

        pltpu.sync_copy(z_hbm, accum.at[pl.ds(sid * ROWS_PER_TILE, ROWS_PER_TILE)])
        plsc.subcore_barrier()

        def issue_gathers(idxref, ii, b):
            pltpu.async_copy(xs_hbm.at[idxref.at[ii, 0]], xsb[b], sg[b])
            pltpu.async_copy(d_hbm.at[idxref.at[ii, 1]], db[b], sg[b])

        def wait_gathers(b):
            pltpu.make_async_copy(xs_hbm.at[pl.ds(0, kch)], xsb[b], sg[b]).wait()
            pltpu.make_async_copy(d_hbm.at[pl.ds(0, kch)], db[b], sg[b]).wait()

        def wait_scatter(b):
            pltpu.make_async_copy(xs_hbm.at[pl.ds(0, kch)], comb[b], ss[b]).wait()

        def wait_iblk(b2):
            pltpu.make_async_copy(eidx_hbm.at[pl.ds(0, BL)], ibs[b2], si[b2]).wait()

        @pl.when(nblk > 0)
        def _():
            pltpu.sync_copy(eidx_hbm.at[pl.ds(cbase, BL)], ib0)
            issue_gathers(ib0, 0, 0)

        def blockpair(t2, carry):
            for b2 in range(2):
                t = 2 * t2 + b2
                ib, ibn = ibs[b2], ibs[1 - b2]
                for i in range(BL):
                    p = i % 2
                    jg = t * BL + i
                    @pl.when(jg >= 1)
                    def _():
                        wait_scatter(1 - p)

                    if i == 0:
                        @pl.when(t + 1 < nblk)
                        def _():
                            pltpu.async_copy(
                                eidx_hbm.at[pl.ds(cbase + (t + 1) * BL, BL)],
                                ibn, si[1 - b2])
                    if i < BL - 1:
                        issue_gathers(ib, i + 1, 1 - p)
                    else:
                        @pl.when(t + 1 < nblk)
                        def _():
                            wait_iblk(1 - b2)
                            issue_gathers(ibn, 0, 1 - p)

                    wait_gathers(p)
                    xsbb, dbb, combb = xsb[p], db[p], comb[p]

                    @plsc.parallel_loop(0, kch, step=1, unroll=4)
                    def _edges(e):
                        sv = xsbb[e, pl.ds(den_col, 16)] + dbb[e, :]
                        w = jnp.exp(jnp.maximum(sv, 0.2 * sv))
                        combb[e, pl.ds(den_col, 16)] = w
                        for g in range(msg_groups):
                            wsc = w[g] if per_group_w else w[0]
                            combb[e, pl.ds(16 * g, 16)] = xsbb[e, pl.ds(16 * g, 16)] * wsc
                    pltpu.async_copy(comb[p], accum.at[ib.at[i, 1]], ss[p], add=True)
            return carry

        lax.fori_loop(0, nblk // 2, blockpair, 0, unroll=False)

        @pl.when(nblk > 0)
        def _():
            wait_scatter(1)
        plsc.subcore_barrier()
        @pl.when(cid == 0)
        def _():
            pltpu.sync_copy(accum.at[pl.ds(sid * ROWS_PER_TILE, ROWS_PER_TILE)],
                            out0_hbm.at[pl.ds(sid * ROWS_PER_TILE, ROWS_PER_TILE)])

        @pl.when(cid == 1)
        def _():
            pltpu.sync_copy(accum.at[pl.ds(sid * ROWS_PER_TILE, ROWS_PER_TILE)],
                            out1_hbm.at[pl.ds(sid * ROWS_PER_TILE, ROWS_PER_TILE)])

    return k(xs, d1, eidx_arr, zrows)


def _tc_combine1(p0, p1, b1, W2p, AS2, AD2):
    def body(p0_ref, p1_ref, b_ref, w_ref, as_ref, ad_ref, xs_ref, d_ref):
        num = p0_ref[...] + p1_ref[...]
        msg = num[:, 0:128]
        recip = 1.0 / (num[:, 128:136] + 1e-16)
        parts = [msg[:, 16 * h:16 * h + 16] * recip[:, h:h + 1] for h in range(8)]
        h1 = jnp.concatenate(parts, axis=1) + b_ref[...]
        h1 = jnp.where(h1 > 0, h1, jnp.exp(jnp.minimum(h1, 0.0)) - 1.0)
        xw2 = jnp.dot(h1, w_ref[...], preferred_element_type=jnp.float32)
        s2 = jnp.dot(xw2, as_ref[...], preferred_element_type=jnp.float32)
        d2 = jnp.dot(xw2, ad_ref[...], preferred_element_type=jnp.float32)
        xs_ref[...] = jnp.concatenate([xw2, s2], axis=1)
        d_ref[...] = d2

    return pl.pallas_call(
        body,
        grid=(8,),
        in_specs=[
            pl.BlockSpec((NPAD // 8, W1ROW), lambda i: (i, 0)),
            pl.BlockSpec((NPAD // 8, W1ROW), lambda i: (i, 0)),
            pl.BlockSpec((1, 128), lambda i: (0, 0)),
            pl.BlockSpec((128, 48), lambda i: (0, 0)),
            pl.BlockSpec((48, 16), lambda i: (0, 0)),
            pl.BlockSpec((48, 16), lambda i: (0, 0)),
        ],
        out_specs=[
            pl.BlockSpec((NPAD // 8, W2ROW), lambda i: (i, 0)),
            pl.BlockSpec((NPAD // 8, 16), lambda i: (i, 0)),
        ],
        out_shape=[
            jax.ShapeDtypeStruct((NPAD, W2ROW), jnp.float32),
            jax.ShapeDtypeStruct((NPAD, 16), jnp.float32),
        ],
    )(p0, p1, b1, W2p, AS2, AD2)


def _tc_final(q0, q1, b2):
    def body(q0_ref, q1_ref, b_ref, o_ref):
        num = q0_ref[...] + q1_ref[...]
        z = num[:, 0:40] / (num[:, 48:49] + 1e-16) + b_ref[...]
        z = jnp.where(z > 0, z, jnp.exp(jnp.minimum(z, 0.0)) - 1.0)
        m = jnp.max(z, axis=1, keepdims=True)
        ez = jnp.exp(z - m)
        ssum = jnp.sum(ez, axis=1, keepdims=True)
        o_ref[...] = z - m - jnp.log(ssum)

    return pl.pallas_call(
        body,
        grid=(8,),
        in_specs=[
            pl.BlockSpec((NPAD // 8, W2ROW), lambda i: (i, 0)),
            pl.BlockSpec((NPAD // 8, W2ROW), lambda i: (i, 0)),
            pl.BlockSpec((1, 40), lambda i: (0, 0)),
        ],
        out_specs=pl.BlockSpec((NPAD // 8, 40), lambda i: (i, 0)),
        out_shape=jax.ShapeDtypeStruct((NPAD, 40), jnp.float32),
    )(q0, q1, b2)


@jax.jit
def kernel(x, edge_index, W1, att_src1, att_dst1, bias1, W2, att_src2, att_dst2, bias2):
    f32 = jnp.float32
    xp = jnp.zeros((NPAD, 128), f32).at[:NN].set(x)
    src = edge_index[0].astype(jnp.int32)
    dst = edge_index[1].astype(jnp.int32)
    padn = jnp.full((EPAD - EE,), NN, jnp.int32)
    srcp = jnp.concatenate([src, padn])
    dstp = jnp.concatenate([dst, padn])

    blkmask = (jnp.arange(128)[:, None] // 16 == jnp.arange(16)[None, :])
    AS = jnp.where(blkmask, att_src1.reshape(128)[:, None], 0.0)
    AD = jnp.where(blkmask, att_dst1.reshape(128)[:, None], 0.0)
    W2p = jnp.pad(W2, ((0, 0), (0, 8)))
    col0 = (jnp.arange(16)[None, :] == 0) & (jnp.arange(48)[:, None] < 40)
    a2pad = jnp.pad(att_src2.reshape(40), (0, 8))
    d2pad = jnp.pad(att_dst2.reshape(40), (0, 8))
    AS2 = jnp.where(col0, a2pad[:, None], 0.0)
    AD2 = jnp.where(col0, d2pad[:, None], 0.0)
    b1 = bias1.reshape(1, 128)
    b2 = bias2.reshape(1, 40)

    z1 = jnp.zeros((ROWS_PER_TILE, W1ROW), f32)
    z2 = jnp.zeros((ROWS_PER_TILE, W2ROW), f32)

    xs1, d1 = _tc_layer1(xp, W1, AS, AD)
    p0, p1 = _sc_edge_phase(xs1, d1, _pack_idx(srcp, dstp, 64), W1ROW, 8, 128,
                            True, 64, z1, 280, 40)
    xs2, d2 = _tc_combine1(p0, p1, b1, W2p, AS2, AD2)
    q0, q1 = _sc_edge_phase(xs2, d2, _pack_idx(srcp, dstp, 128), W2ROW, 3, 48,
                            False, 128, z2, 136, 24)
    out = _tc_final(q0, q1, b2)
    return out[:NN]
